# baseline (speedup 1.0000x reference)
import functools

import jax
import jax.numpy as jnp
from jax import lax
from jax.experimental import pallas as pl
from jax.experimental.pallas import tpu as pltpu
from jax.experimental.pallas import tpu_sc as plsc

B = 4
S = 8192
D = 768
L = 16
NC = 2
NS = 16
NW = NC * NS
POS_W = S // NW
PP = 8
K = PP * B
NCHUNK = POS_W // PP
GROUPS = D // L

_MESH = plsc.VectorSubcoreMesh(
    core_axis_name="c", subcore_axis_name="s", num_cores=NC, num_subcores=NS
)


@functools.partial(
    pl.kernel,
    out_type=jax.ShapeDtypeStruct((B * S, D), jnp.float32),
    mesh=_MESH,
    scratch_types=[
        pltpu.VMEM((NCHUNK, K), jnp.int32),
        pltpu.VMEM((K, D), jnp.float32),
        pltpu.VMEM((K, D), jnp.float32),
        pltpu.VMEM((PP, D), jnp.float32),
        pltpu.VMEM((PP, D), jnp.float32),
        pltpu.VMEM((K, D), jnp.float32),
        pltpu.VMEM((K, D), jnp.float32),
        pltpu.SemaphoreType.DMA,
        pltpu.SemaphoreType.DMA,
        pltpu.SemaphoreType.DMA,
        pltpu.SemaphoreType.DMA,
        pltpu.SemaphoreType.DMA,
        pltpu.SemaphoreType.DMA,
    ],
)
def _emb_kernel(x_hbm, table_hbm, pe_hbm, out_hbm,
                idx_v, tok0, tok1, pe0, pe1, res0, res1,
                sg0, sg1, sp0, sp1, so0, so1):
    wid = lax.axis_index("s") * NC + lax.axis_index("c")
    pos0 = wid * POS_W

    toks = (tok0, tok1)
    pes = (pe0, pe1)
    ress = (res0, res1)
    sgs = (sg0, sg1)
    sps = (sp0, sp1)
    sos = (so0, so1)

    pltpu.sync_copy(x_hbm.at[pl.ds(wid * NCHUNK, NCHUNK)], idx_v)

    def start_chunk(j, b):
        pltpu.async_copy(table_hbm.at[idx_v.at[j]], toks[b], sgs[b])
        pltpu.async_copy(pe_hbm.at[pl.ds(pos0 + j * PP, PP)], pes[b], sps[b])

    def start_out(j, b):
        for bb in range(B):
            pltpu.async_copy(
                ress[b].at[pl.ds(bb * PP, PP)],
                out_hbm.at[pl.ds(bb * S + pos0 + j * PP, PP)], sos[b])

    def wait_out(b):
        for _ in range(B):
            pltpu.make_async_copy(
                ress[b].at[pl.ds(0, PP)], out_hbm.at[pl.ds(0, PP)],
                sos[b]).wait()

    def lane_bcast(v, lane_idx):
        return lax.gather(
            v, jnp.full((L, 1), lane_idx, jnp.int32),
            dimension_numbers=lax.GatherDimensionNumbers(
                offset_dims=(), collapsed_slice_dims=(0,),
                start_index_map=(0,)),
            slice_sizes=(1,),
            mode=lax.GatherScatterMode.PROMISE_IN_BOUNDS)

    start_chunk(0, 0)

    def loop_body(jj, _):
        for b in range(2):
            j = jj * 2 + b
            nb = 1 - b

            @pl.when(j + 1 < NCHUNK)
            def _():
                @pl.when(j >= 1)
                def _():
                    wait_out(nb)
                start_chunk(j + 1, nb)

            pltpu.make_async_copy(
                table_hbm.at[idx_v.at[j]], toks[b], sgs[b]).wait()
            pltpu.make_async_copy(
                pe_hbm.at[pl.ds(pos0, PP)], pes[b], sps[b]).wait()

            def pos_body(i, _):
                iv_lo = idx_v[j, pl.ds(0, L)]
                iv_hi = idx_v[j, pl.ds(L, L)]
                mv_lo = jnp.where(iv_lo != 0, 1.0, 0.0).astype(jnp.float32)
                mv_hi = jnp.where(iv_hi != 0, 1.0, 0.0).astype(jnp.float32)
                masks = (lane_bcast(mv_lo, i),
                         lane_bcast(mv_lo, i + PP),
                         lane_bcast(mv_hi, i),
                         lane_bcast(mv_hi, i + PP))
                for g in range(GROUPS):
                    sl = pl.ds(g * L, L)
                    pe_g = pes[b][i, sl]
                    for bb in range(B):
                        r = bb * PP + i
                        ress[b][r, sl] = toks[b][r, sl] * masks[bb] + pe_g
                return 0

            lax.fori_loop(0, PP, pos_body, 0)

            start_out(j, b)
        return 0

    lax.fori_loop(0, NCHUNK // 2, loop_body, 0)

    wait_out(0)
    wait_out(1)


def kernel(x, table, pe):
    xp = (x.astype(jnp.int32)
          .reshape(B, NW, NCHUNK, PP)
          .transpose(1, 2, 0, 3)
          .reshape(NW * NCHUNK, K))
    out = _emb_kernel(xp, table, pe)
    return out.reshape(B, S, D)
